# Initial kernel scaffold; baseline (speedup 1.0000x reference)
import jax
import jax.numpy as jnp
from jax.experimental import pallas as pl


def kernel(head, relation, tail, timestamp, entity_table, relation_table, timestamp_table):
    raise NotImplementedError("write your pallas kernel here")


# speedup vs baseline: 1.3675x; 1.3675x over previous
import functools

import jax
import jax.numpy as jnp
from jax import lax
from jax.experimental import pallas as pl
from jax.experimental.pallas import tpu as pltpu
from jax.experimental.pallas import tpu_sc as plsc

NUM_ENTITIES = 100000
NUM_RELATIONS = 64
NUM_TIMESTAMPS = 1024
EMBED_DIM = 64
BATCH = 16384

NC = 2
NS = 16
NW = NC * NS
B_PER_W = BATCH // NW
CHUNK = 128
NCHUNK = B_PER_W // CHUNK
NOPS = 4
NTASK = NOPS * NCHUNK
NBUF = 4

_MESH = plsc.VectorSubcoreMesh(core_axis_name="c", subcore_axis_name="s")

_OUT = tuple(
    jax.ShapeDtypeStruct((BATCH, EMBED_DIM), jnp.float32) for _ in range(NOPS)
)

_SCRATCH = (
    [pltpu.VMEM((NTASK, CHUNK), jnp.int32)]
    + [pltpu.VMEM((CHUNK, EMBED_DIM), jnp.float32) for _ in range(NBUF)]
    + [pltpu.SemaphoreType.DMA for _ in range(2 * NBUF)]
)


@functools.partial(
    pl.kernel,
    out_type=_OUT,
    mesh=_MESH,
    scratch_types=_SCRATCH,
    compiler_params=pltpu.CompilerParams(use_tc_tiling_on_sc=False),
)
def _tkg_gather(h2, r2, t2, s2, ent_tbl, rel_tbl, ts_tbl,
                out_h, out_r, out_t, out_s, idx_v,
                b0, b1, b2, b3,
                g0, g1, g2, g3, s0, s1, s2s, s3):
    wid = lax.axis_index("s") * NC + lax.axis_index("c")
    rowbase = wid * NCHUNK
    outbase = wid * B_PER_W

    bufs = [b0, b1, b2, b3]
    gsems = [g0, g1, g2, g3]
    ssems = [s0, s1, s2s, s3]
    idx_srcs = [h2, r2, t2, s2]
    tables = [ent_tbl, rel_tbl, ent_tbl, ts_tbl]
    outs = [out_h, out_r, out_t, out_s]

    for op in range(NOPS):
        pltpu.sync_copy(idx_srcs[op].at[pl.ds(rowbase, NCHUNK)],
                        idx_v.at[pl.ds(op * NCHUNK, NCHUNK)])

    tasks = [
        (tables[op], outs[op], op * NCHUNK + c, c * CHUNK)
        for op in range(NOPS)
        for c in range(NCHUNK)
    ]

    gcopies = [None] * NTASK
    scopies = [None] * NTASK

    def start_store(p):
        _, out_p, _, off_p = tasks[p]
        gcopies[p].wait()
        scopies[p] = pltpu.async_copy(
            bufs[p % NBUF], out_p.at[pl.ds(outbase + off_p, CHUNK)],
            ssems[p % NBUF])

    for step in range(NTASK):
        buf = step % NBUF
        if step >= NBUF:
            scopies[step - NBUF].wait()
        tbl, _, irow, _ = tasks[step]
        gcopies[step] = pltpu.async_copy(
            tbl.at[idx_v.at[irow]], bufs[buf], gsems[buf])
        if step >= 1:
            start_store(step - 1)

    start_store(NTASK - 1)
    for p in range(NTASK - NBUF, NTASK):
        scopies[p].wait()


def kernel(head, relation, tail, timestamp,
           entity_table, relation_table, timestamp_table):
    shp = (BATCH // CHUNK, CHUNK)
    h2 = head.astype(jnp.int32).reshape(shp)
    r2 = relation.astype(jnp.int32).reshape(shp)
    t2 = tail.astype(jnp.int32).reshape(shp)
    s2 = timestamp.astype(jnp.int32).reshape(shp)
    return _tkg_gather(h2, r2, t2, s2,
                       entity_table, relation_table, timestamp_table)


# speedup vs baseline: 1.4077x; 1.0294x over previous
import functools

import jax
import jax.numpy as jnp
from jax import lax
from jax.experimental import pallas as pl
from jax.experimental.pallas import tpu as pltpu
from jax.experimental.pallas import tpu_sc as plsc

NUM_ENTITIES = 100000
NUM_RELATIONS = 64
NUM_TIMESTAMPS = 1024
EMBED_DIM = 64
BATCH = 16384

NC = 2
NS = 16
NW = NC * NS
B_PER_W = BATCH // NW
CHUNK = 128
NCHUNK = B_PER_W // CHUNK
NOPS = 4
NTASK = NOPS * NCHUNK
NBUF = 8

_MESH = plsc.VectorSubcoreMesh(core_axis_name="c", subcore_axis_name="s")

_OUT = tuple(
    jax.ShapeDtypeStruct((BATCH, EMBED_DIM), jnp.float32) for _ in range(NOPS)
)

_SCRATCH = (
    [pltpu.VMEM((NTASK, CHUNK), jnp.int32)]
    + [pltpu.VMEM((CHUNK, EMBED_DIM), jnp.float32) for _ in range(NBUF)]
    + [pltpu.SemaphoreType.DMA for _ in range(2 * NBUF + NOPS)]
)


@functools.partial(
    pl.kernel,
    out_type=_OUT,
    mesh=_MESH,
    scratch_types=_SCRATCH,
    compiler_params=pltpu.CompilerParams(use_tc_tiling_on_sc=False),
)
def _tkg_gather(h2, r2, t2, s2, ent_tbl, rel_tbl, ts_tbl,
                out_h, out_r, out_t, out_s, idx_v,
                b0, b1, b2, b3, b4, b5, b6, b7,
                g0, g1, g2, g3, g4, g5, g6, g7,
                s0, s1, s2s, s3, s4, s5, s6, s7,
                i0, i1, i2, i3):
    wid = lax.axis_index("s") * NC + lax.axis_index("c")
    rowbase = wid * NCHUNK
    outbase = wid * B_PER_W

    bufs = [b0, b1, b2, b3, b4, b5, b6, b7]
    gsems = [g0, g1, g2, g3, g4, g5, g6, g7]
    ssems = [s0, s1, s2s, s3, s4, s5, s6, s7]
    isems = [i0, i1, i2, i3]
    idx_srcs = [h2, r2, t2, s2]
    tables = [ent_tbl, rel_tbl, ent_tbl, ts_tbl]
    outs = [out_h, out_r, out_t, out_s]

    icopies = []
    for op in range(NOPS):
        icopies.append(pltpu.async_copy(
            idx_srcs[op].at[pl.ds(rowbase, NCHUNK)],
            idx_v.at[pl.ds(op * NCHUNK, NCHUNK)], isems[op]))
    idx_ready = [False] * NOPS

    tasks = [
        (tables[op], outs[op], op, op * NCHUNK + c, c * CHUNK)
        for op in range(NOPS)
        for c in range(NCHUNK)
    ]

    gcopies = [None] * NTASK
    scopies = [None] * NTASK

    def start_gather(p):
        tbl, _, op, irow, _ = tasks[p]
        if not idx_ready[op]:
            icopies[op].wait()
            idx_ready[op] = True
        gcopies[p] = pltpu.async_copy(
            tbl.at[idx_v.at[irow]], bufs[p % NBUF], gsems[p % NBUF])

    def start_store(p):
        _, out_p, _, _, off_p = tasks[p]
        gcopies[p].wait()
        scopies[p] = pltpu.async_copy(
            bufs[p % NBUF], out_p.at[pl.ds(outbase + off_p, CHUNK)],
            ssems[p % NBUF])

    for step in range(NTASK):
        if step >= NBUF:
            scopies[step - NBUF].wait()
        start_gather(step)
        d = step - (NBUF - 1)
        if d >= 0:
            start_store(d)
    for d in range(NTASK - NBUF + 1, NTASK):
        start_store(d)
    for d in range(NTASK - NBUF, NTASK):
        scopies[d].wait()


def kernel(head, relation, tail, timestamp,
           entity_table, relation_table, timestamp_table):
    shp = (BATCH // CHUNK, CHUNK)
    h2 = head.astype(jnp.int32).reshape(shp)
    r2 = relation.astype(jnp.int32).reshape(shp)
    t2 = tail.astype(jnp.int32).reshape(shp)
    s2 = timestamp.astype(jnp.int32).reshape(shp)
    return _tkg_gather(h2, r2, t2, s2,
                       entity_table, relation_table, timestamp_table)


# speedup vs baseline: 1.6484x; 1.1710x over previous
import functools

import jax
import jax.numpy as jnp
from jax import lax
from jax.experimental import pallas as pl
from jax.experimental.pallas import tpu as pltpu
from jax.experimental.pallas import tpu_sc as plsc

NUM_ENTITIES = 100000
NUM_RELATIONS = 64
NUM_TIMESTAMPS = 1024
EMBED_DIM = 64
BATCH = 16384

NC = 2
NS = 16
NW = NC * NS
B_PER_W = BATCH // NW
CHUNK = 128
NCHUNK = B_PER_W // CHUNK
NOPS = 4
NTASK = NOPS * NCHUNK
NBUF = 4
PADDED = 128

_MESH = plsc.VectorSubcoreMesh(core_axis_name="c", subcore_axis_name="s")

_OUT = tuple(
    jax.ShapeDtypeStruct((8, BATCH // CHUNK * 8, CHUNK), jnp.float32)
    for _ in range(NOPS)
)

_SCRATCH = (
    [pltpu.VMEM((NTASK, CHUNK), jnp.int32)]
    + [pltpu.VMEM((CHUNK, PADDED), jnp.float32) for _ in range(NBUF)]
    + [pltpu.VMEM((8, 8, 129), jnp.float32) for _ in range(NBUF)]
    + [pltpu.SemaphoreType.DMA for _ in range(2 * NBUF + NOPS)]
)


@functools.partial(
    pl.kernel,
    out_type=_OUT,
    mesh=_MESH,
    scratch_types=_SCRATCH,
    compiler_params=pltpu.CompilerParams(use_tc_tiling_on_sc=False, needs_layout_passes=False),
)
def _tkg_gather(h2, r2, t2, s2, ent_tbl, rel_tbl, ts_tbl,
                out_h, out_r, out_t, out_s, idx_v,
                b0, b1, b2, b3,
                t0, t1, t2b, t3,
                g0, g1, g2, g3,
                s0, s1, s2s, s3,
                i0, i1, i2, i3):
    wid = lax.axis_index("s") * NC + lax.axis_index("c")
    rowbase = wid * NCHUNK
    outbase = wid * B_PER_W

    bufs = [b0, b1, b2, b3]
    tbufs = [t0, t1, t2b, t3]
    gsems = [g0, g1, g2, g3]
    ssems = [s0, s1, s2s, s3]
    isems = [i0, i1, i2, i3]
    idx_srcs = [h2, r2, t2, s2]
    tables = [ent_tbl, rel_tbl, ent_tbl, ts_tbl]
    outs = [out_h, out_r, out_t, out_s]

    icopies = []
    for op in range(NOPS):
        icopies.append(pltpu.async_copy(
            idx_srcs[op].at[pl.ds(rowbase, NCHUNK)],
            idx_v.at[pl.ds(op * NCHUNK, NCHUNK)], isems[op]))
    idx_ready = [False] * NOPS

    tasks = [
        (tables[op], outs[op], op, op * NCHUNK + c, c * CHUNK)
        for op in range(NOPS)
        for c in range(NCHUNK)
    ]

    gcopies = [None] * NTASK
    scopies = [None] * NTASK

    def start_gather(p):
        tbl, _, op, irow, _ = tasks[p]
        if not idx_ready[op]:
            icopies[op].wait()
            idx_ready[op] = True
        gcopies[p] = pltpu.async_copy(
            tbl.at[idx_v.at[irow]], bufs[p % NBUF], gsems[p % NBUF])

    iota16 = lax.iota(jnp.int32, 16)

    def transpose_chunk(buf, tbuf):
        def body(c, carry):
            cc = jnp.full((16,), c, jnp.int32)
            for d0 in range(0, EMBED_DIM, 16):
                dvec = d0 + iota16
                v = plsc.load_gather(buf, [cc, dvec])
                plsc.store_scatter(tbuf, [dvec // 8, dvec % 8, cc], v)
            return carry
        lax.fori_loop(0, CHUNK, body, jnp.int32(0))

    def start_store(p):
        _, out_p, _, _, off_p = tasks[p]
        gcopies[p].wait()
        transpose_chunk(bufs[p % NBUF], tbufs[p % NBUF])
        tcg = wid * NCHUNK + off_p // CHUNK
        scopies[p] = pltpu.async_copy(
            tbufs[p % NBUF].at[:, :, pl.ds(0, CHUNK)],
            out_p.at[:, pl.ds(tcg * 8, 8), :],
            ssems[p % NBUF])

    for step in range(NTASK):
        if step >= NBUF:
            scopies[step - NBUF].wait()
        start_gather(step)
        d = step - (NBUF - 1)
        if d >= 0:
            start_store(d)
    for d in range(NTASK - NBUF + 1, NTASK):
        start_store(d)
    for d in range(NTASK - NBUF, NTASK):
        scopies[d].wait()


def kernel(head, relation, tail, timestamp,
           entity_table, relation_table, timestamp_table):
    shp = (BATCH // CHUNK, CHUNK)
    h2 = head.astype(jnp.int32).reshape(shp)
    r2 = relation.astype(jnp.int32).reshape(shp)
    t2 = tail.astype(jnp.int32).reshape(shp)
    s2 = timestamp.astype(jnp.int32).reshape(shp)
    pad = ((0, 0), (0, PADDED - EMBED_DIM))
    ent_p = jnp.pad(entity_table, pad)
    rel_p = jnp.pad(relation_table, pad)
    ts_p = jnp.pad(timestamp_table, pad)
    outs = _tkg_gather(h2, r2, t2, s2, ent_p, rel_p, ts_p)

    def _untile(o):
        o4 = o.reshape(8, BATCH // CHUNK, 8, CHUNK)
        return o4.transpose(1, 3, 0, 2).reshape(BATCH, EMBED_DIM)

    return tuple(_untile(o) for o in outs)
